# baseline (speedup 1.0000x reference)
import functools

import jax
import jax.numpy as jnp
import numpy as np
from jax.experimental import pallas as pl
from jax.experimental.pallas import tpu as pltpu

_NPOINTS = 2048


def _fps_body(npoints, start_ref, x_ref, y_ref, z_ref, o_ref, dist_ref):
    B, N = x_ref.shape
    W = 256
    NB = N // W
    LEVELS = NB.bit_length() - 1
    colW = jax.lax.broadcasted_iota(jnp.int32, (B, W), 1)
    col128 = jax.lax.broadcasted_iota(jnp.int32, (B, 128), 1)

    dist_ref[...] = jnp.full((B, N), jnp.inf, dtype=jnp.float32)

    def gather(cur):
        blk = cur // W
        bits = [((blk >> l) & 1) == 1 for l in range(LEVELS)]

        def pick(ref, lo, level):
            if level < 0:
                return ref[:, pl.ds(lo * W, W)]
            half = 1 << level
            lo_v = pick(ref, lo, level - 1)
            hi_v = pick(ref, lo + half, level - 1)
            return jnp.where(bits[level], hi_v, lo_v)

        lane_oh = colW == (cur % W)

        def extract(ref):
            sel = pick(ref, 0, LEVELS - 1)
            return jnp.sum(jnp.where(lane_oh, sel, 0.0), axis=1,
                           keepdims=True)

        return extract(x_ref), extract(y_ref), extract(z_ref)

    def step(t, cur):
        px, py, pz = gather(cur)

        base = pl.multiple_of((t // 128) * 128, 128)
        tile = o_ref[:, pl.ds(base, 128)]
        o_ref[:, pl.ds(base, 128)] = jnp.where(col128 == t - base, cur, tile)

        bd = jnp.full((B, W), -jnp.inf, jnp.float32)
        bk = jnp.zeros((B, W), jnp.int32)
        for k in range(NB):
            sl = pl.ds(k * W, W)
            dx = x_ref[:, sl] - px
            dy = y_ref[:, sl] - py
            dz = z_ref[:, sl] - pz
            d = (dx * dx + dz * dz) + dy * dy
            dd = jnp.minimum(dist_ref[:, sl], d)
            dist_ref[:, sl] = dd
            take = dd > bd
            bd = jnp.where(take, dd, bd)
            bk = jnp.where(take, k, bk)

        m = jnp.max(bd, axis=1, keepdims=True)
        cand = jnp.where(bd == m, bk * W + colW, N)
        nxt = jnp.min(cand, axis=1, keepdims=True)
        return nxt

    def step4(i, cur):
        cur = step(4 * i + 1, step(4 * i, cur))
        return step(4 * i + 3, step(4 * i + 2, cur))

    jax.lax.fori_loop(0, npoints // 4, step4, start_ref[...])


@functools.partial(jax.jit, static_argnames=("npoints",))
def _fps_pallas(x, y, z, start, npoints):
    B, N = x.shape
    out = pl.pallas_call(
        functools.partial(_fps_body, npoints),
        out_shape=jax.ShapeDtypeStruct((B, npoints), jnp.int32),
        scratch_shapes=[pltpu.VMEM((B, N), jnp.float32)],
    )(start, x, y, z)
    return out


def kernel(pos):
    B, N, C = pos.shape
    start_idx = jax.random.randint(
        jax.random.key(42), (B,), 0, N - 1, dtype=jnp.int32
    ).reshape(B, 1)
    x = pos[:, :, 0]
    y = pos[:, :, 1]
    z = pos[:, :, 2]

    ndev = jax.device_count()
    if ndev >= 2 and B % 2 == 0:
        mesh = jax.sharding.Mesh(np.array(jax.devices()[:2]), ("b",))
        P = jax.sharding.PartitionSpec
        try:
            smap = jax.shard_map
        except AttributeError:
            from jax.experimental.shard_map import shard_map as smap
        f = smap(
            lambda xs, ys, zs, ss: _fps_pallas(xs, ys, zs, ss, _NPOINTS),
            mesh=mesh,
            in_specs=(P("b", None),) * 4,
            out_specs=P("b", None),
            check_vma=False,
        )
        res = f(x, y, z, start_idx)
    else:
        res = _fps_pallas(x, y, z, start_idx, _NPOINTS)
    return res.reshape(-1).astype(jnp.int32)


# speedup vs baseline: 1.0529x; 1.0529x over previous
import functools

import jax
import jax.numpy as jnp
import numpy as np
from jax.experimental import pallas as pl
from jax.experimental.pallas import tpu as pltpu

_NPOINTS = 2048


def _fps_body(npoints, start_ref, x_ref, y_ref, z_ref, o_ref, dist_ref):
    B, N = x_ref.shape
    W = 256
    NB = N // W
    LEVELS = NB.bit_length() - 1
    colW = jax.lax.broadcasted_iota(jnp.int32, (B, W), 1)
    col128 = jax.lax.broadcasted_iota(jnp.int32, (B, 128), 1)

    dist_ref[...] = jnp.full((B, N), jnp.inf, dtype=jnp.float32)

    def gather(cur):
        blk = cur // W
        bits = [((blk >> l) & 1) == 1 for l in range(LEVELS)]

        def pick(ref, lo, level):
            if level < 0:
                return ref[:, pl.ds(lo * W, W)]
            half = 1 << level
            lo_v = pick(ref, lo, level - 1)
            hi_v = pick(ref, lo + half, level - 1)
            return jnp.where(bits[level], hi_v, lo_v)

        lane_oh = colW == (cur % W)

        def extract(ref):
            sel = pick(ref, 0, LEVELS - 1)
            return jnp.sum(jnp.where(lane_oh, sel, 0.0), axis=1,
                           keepdims=True)

        return extract(x_ref), extract(y_ref), extract(z_ref)

    def step(t, cur):
        px, py, pz = gather(cur)

        base = pl.multiple_of((t // 128) * 128, 128)
        tile = o_ref[:, pl.ds(base, 128)]
        o_ref[:, pl.ds(base, 128)] = jnp.where(col128 == t - base, cur, tile)

        bd = jnp.full((B, W), -jnp.inf, jnp.float32)
        bk = jnp.zeros((B, W), jnp.int32)
        for k in range(NB):
            sl = pl.ds(k * W, W)
            dx = x_ref[:, sl] - px
            dy = y_ref[:, sl] - py
            dz = z_ref[:, sl] - pz
            d = (dx * dx + dz * dz) + dy * dy
            dd = jnp.minimum(dist_ref[:, sl], d)
            dist_ref[:, sl] = dd
            take = dd > bd
            bd = jnp.where(take, dd, bd)
            bk = jnp.where(take, k, bk)

        m = jnp.max(bd, axis=1, keepdims=True)
        cand = jnp.where(bd == m, bk * W + colW, N)
        nxt = jnp.min(cand, axis=1, keepdims=True)
        return nxt

    def step4(i, cur):
        cur = step(4 * i + 1, step(4 * i, cur))
        return step(4 * i + 3, step(4 * i + 2, cur))

    jax.lax.fori_loop(0, npoints // 4, step4, start_ref[...])


@functools.partial(jax.jit, static_argnames=("npoints",))
def _fps_pallas(x, y, z, start, npoints):
    B, N = x.shape
    out = pl.pallas_call(
        functools.partial(_fps_body, npoints),
        out_shape=jax.ShapeDtypeStruct((B, npoints), jnp.int32),
        scratch_shapes=[pltpu.VMEM((B, N), jnp.float32)],
    )(start, x, y, z)
    return out


def kernel(pos):
    B, N, C = pos.shape
    start_idx = jax.random.randint(
        jax.random.key(42), (B,), 0, N - 1, dtype=jnp.int32
    ).reshape(B, 1)
    def run(pos_s, ss):
        xs = pos_s[:, :, 0]
        ys = pos_s[:, :, 1]
        zs = pos_s[:, :, 2]
        return _fps_pallas(xs, ys, zs, ss, _NPOINTS)

    ndev = jax.device_count()
    if ndev >= 2 and B % 2 == 0:
        mesh = jax.sharding.Mesh(np.array(jax.devices()[:2]), ("b",))
        P = jax.sharding.PartitionSpec
        try:
            smap = jax.shard_map
        except AttributeError:
            from jax.experimental.shard_map import shard_map as smap
        f = smap(
            run,
            mesh=mesh,
            in_specs=(P("b", None, None), P("b", None)),
            out_specs=P("b", None),
            check_vma=False,
        )
        res = f(pos, start_idx)
    else:
        res = run(pos, start_idx)
    return res.reshape(-1).astype(jnp.int32)
